# baseline (speedup 1.0000x reference)
import numpy as np
import jax
import jax.numpy as jnp
from jax.experimental import pallas as pl
from jax.experimental.pallas import tpu as pltpu

_SHIFTS = (0, 1, 3)
_G = len(_SHIFTS)
_BLK = 384
_SRC = 128
_K1 = 136
_ROWS = 256


def _one_hot_e(n_features, k_rows, n_cols, j0):
    e = np.zeros((k_rows, n_cols), np.float32)
    for b in range(n_cols):
        a = b // _G + _SHIFTS[b % _G]
        if a < k_rows and j0 + a < n_features:
            e[a, b] = 1.0
    return jnp.asarray(e, jnp.bfloat16)


def _tok_kernel(e1_ref, e2_ref, x_ref, o_ref, *, n, nt_full, tail_j0):
    x = x_ref[...]
    xh = x.astype(jnp.bfloat16)
    xl = (x - xh.astype(jnp.float32)).astype(jnp.bfloat16)
    e1 = e1_ref[...]
    for t in range(nt_full):
        j0 = t * _SRC
        acc = jnp.dot(xh[:, j0:j0 + _K1], e1,
                      preferred_element_type=jnp.float32)
        acc += jnp.dot(xl[:, j0:j0 + _K1], e1,
                       preferred_element_type=jnp.float32)
        o_ref[:, t * _BLK:(t + 1) * _BLK] = acc
    e2 = e2_ref[...]
    acc = jnp.dot(xh[:, tail_j0:n], e2, preferred_element_type=jnp.float32)
    acc += jnp.dot(xl[:, tail_j0:n], e2, preferred_element_type=jnp.float32)
    o_ref[:, nt_full * _BLK:] = acc
    for c in range(_G * n - _G * max(_SHIFTS), _G * n):
        j, s = c // _G, _SHIFTS[c % _G]
        if j + s >= n:
            o_ref[:, pl.ds(c, 1)] = x_ref[:, pl.ds((j + s) % n, 1)]


def kernel(x_all):
    b, n = x_all.shape
    out_cols = _G * n
    nt_full = out_cols // _BLK
    tail_j0 = nt_full * _SRC
    tail_n = out_cols - nt_full * _BLK
    e1 = _one_hot_e(n, _K1, _BLK, 0)
    e2 = _one_hot_e(n, n - tail_j0, tail_n, tail_j0)

    import functools
    body = functools.partial(_tok_kernel, n=n, nt_full=nt_full,
                             tail_j0=tail_j0)
    out = pl.pallas_call(
        body,
        grid=(b // _ROWS,),
        in_specs=[
            pl.BlockSpec(e1.shape, lambda i: (0, 0)),
            pl.BlockSpec(e2.shape, lambda i: (0, 0)),
            pl.BlockSpec((_ROWS, n), lambda i: (i, 0)),
        ],
        out_specs=pl.BlockSpec((_ROWS, out_cols), lambda i: (i, 0)),
        out_shape=jax.ShapeDtypeStruct((b, out_cols), x_all.dtype),
        compiler_params=pltpu.CompilerParams(
            dimension_semantics=("parallel",),
        ),
    )(e1, e2, x_all)
    return out.reshape(b, n, _G)


# speedup vs baseline: 15.1168x; 15.1168x over previous
import jax
import jax.numpy as jnp
from jax.experimental import pallas as pl
from jax.experimental.pallas import tpu as pltpu

_SHIFTS = (0, 1, 3)
_COLS = 512


def _tok_kernel(x_ref, o_ref):
    x = x_ref[...]
    for k, s in enumerate(_SHIFTS):
        o_ref[k] = jnp.concatenate([x[s:], x[:s]], axis=0) if s else x


def kernel(x_all):
    b, n = x_all.shape
    g = len(_SHIFTS)
    xt = x_all.T
    yt = pl.pallas_call(
        _tok_kernel,
        grid=(b // _COLS,),
        in_specs=[pl.BlockSpec((n, _COLS), lambda i: (0, i))],
        out_specs=pl.BlockSpec((g, n, _COLS), lambda i: (0, 0, i)),
        out_shape=jax.ShapeDtypeStruct((g, n, b), x_all.dtype),
        compiler_params=pltpu.CompilerParams(
            dimension_semantics=("parallel",),
        ),
    )(xt)
    return yt.transpose(2, 1, 0)


# speedup vs baseline: 15.4107x; 1.0194x over previous
import jax
import jax.numpy as jnp
from jax.experimental import pallas as pl
from jax.experimental.pallas import tpu as pltpu

_SHIFTS = (0, 1, 3)
_COLS = 1024


def _tok_kernel(x_ref, o_ref):
    x = x_ref[...]
    for k, s in enumerate(_SHIFTS):
        o_ref[k] = jnp.concatenate([x[s:], x[:s]], axis=0) if s else x


def kernel(x_all):
    b, n = x_all.shape
    g = len(_SHIFTS)
    xt = x_all.T
    yt = pl.pallas_call(
        _tok_kernel,
        grid=(b // _COLS,),
        in_specs=[pl.BlockSpec((n, _COLS), lambda i: (0, i))],
        out_specs=pl.BlockSpec((g, n, _COLS), lambda i: (0, 0, i)),
        out_shape=jax.ShapeDtypeStruct((g, n, b), x_all.dtype),
        compiler_params=pltpu.CompilerParams(
            dimension_semantics=("parallel",),
        ),
    )(xt)
    return yt.transpose(2, 1, 0)
